# baseline (speedup 1.0000x reference)
import functools

import jax
import jax.numpy as jnp
from jax import lax
from jax.experimental import pallas as pl
from jax.experimental.pallas import tpu as pltpu
from jax.experimental.pallas import tpu_sc as plsc

_BS, _L, _D = 128, 12, 8192
_NW = 32
_CB = 8
_HL = 4096
_UNITS = _L * (_BS // _CB) * (_D // _HL)
_NU = _UNITS // _NW
_RB = 3


def _sc_body(in_hbm, idx_hbm, drop_hbm, out_hbm, idx_v, drop_v, buf, in_sems, out_sems):
    cid = lax.axis_index("c")
    sid = lax.axis_index("s")
    w = sid * 2 + cid

    pltpu.sync_copy(idx_hbm, idx_v)
    pltpu.sync_copy(drop_hbm, drop_v)

    lane = lax.broadcasted_iota(jnp.int32, (16,), 0)

    def unit(i):
        u = i * _NW + w
        l = u // 32
        c = u % 32
        b0 = (c % 16) * _CB
        h = c // 16
        return l, b0, h

    def _in(i):
        l, b0, h = unit(i)
        return pltpu.make_async_copy(
            in_hbm.at[l, pl.ds(b0, _CB), pl.ds(h * _HL, _HL)],
            buf.at[i % _RB],
            in_sems.at[i % _RB],
        )

    def _out(i):
        l, b0, h = unit(i)
        return pltpu.make_async_copy(
            buf.at[i % _RB],
            out_hbm.at[l, pl.ds(b0, _CB), pl.ds(h * _HL, _HL)],
            out_sems.at[i % _RB],
        )

    def _fix(i):
        l, b0, _ = unit(i)
        slot = i % _RB
        idx16 = idx_v[pl.ds((b0 // 16) * 16, 16)]
        drop16 = drop_v[pl.ds((b0 // 16) * 16, 16)]
        flags16 = jnp.where(
            jnp.logical_and(drop16 > (1.0 - 0.2), idx16 == l), 1, 0
        )
        z = jnp.zeros((16,), jnp.float32)
        for j in range(_CB):
            sel = lane == ((b0 + j) % 16)
            flag = jnp.max(jnp.where(sel, flags16, 0))

            @pl.when(flag > 0)
            def _():
                def body(t, _):
                    buf[slot, j, pl.ds(t * 16, 16)] = z
                    return 0

                lax.fori_loop(0, _HL // 16, body, 0)

    _in(0).start()
    for i in range(_NU):
        if i + 1 < _NU:
            if i + 1 >= _RB:
                _out(i + 1 - _RB).wait()
            _in(i + 1).start()
        _in(i).wait()
        _fix(i)
        _out(i).start()
    for i in range(_NU - _RB, _NU):
        _out(i).wait()


def _sc_call(x_t, index, drop_rand):
    mesh = plsc.VectorSubcoreMesh(core_axis_name="c", subcore_axis_name="s")
    k = functools.partial(
        pl.kernel,
        mesh=mesh,
        compiler_params=pltpu.CompilerParams(needs_layout_passes=False),
        out_type=jax.ShapeDtypeStruct((_L, _BS, _D), jnp.float32),
        scratch_types=[
            pltpu.VMEM((_BS,), jnp.int32),
            pltpu.VMEM((_BS,), jnp.float32),
            pltpu.VMEM((_RB, _CB, _HL), jnp.float32),
            pltpu.SemaphoreType.DMA((_RB,)),
            pltpu.SemaphoreType.DMA((_RB,)),
        ],
    )(_sc_body)
    return k(x_t, index, drop_rand)


@jax.jit
def kernel(inputs, index, drop_rand):
    x_t = jnp.transpose(inputs, (1, 0, 2))
    out_t = _sc_call(x_t, index, drop_rand)
    return jnp.transpose(out_t, (1, 0, 2))


# speedup vs baseline: 1.6087x; 1.6087x over previous
import jax
import jax.numpy as jnp
from jax.experimental import pallas as pl
from jax.experimental.pallas import tpu as pltpu

_BS, _L, _D = 128, 12, 8192


def _body(idx_ref, drop_ref, in_ref, out_ref):
    l0 = pl.program_id(0) * 2
    out_ref[...] = in_ref[...]

    def patch(b, _):
        dropped = drop_ref[b] > (1.0 - 0.2)
        for k in range(2):

            @pl.when(jnp.logical_and(dropped, idx_ref[b] == l0 + k))
            def _():
                out_ref[k, pl.ds(b, 1), :] = jnp.zeros((1, _D), jnp.float32)

        return 0

    jax.lax.fori_loop(0, _BS, patch, 0)


def _transposed_call(index, drop_rand, x_t):
    grid_spec = pltpu.PrefetchScalarGridSpec(
        num_scalar_prefetch=2,
        grid=(_L // 2,),
        in_specs=[
            pl.BlockSpec((2, _BS, _D), lambda l, idx_ref, drop_ref: (l, 0, 0)),
        ],
        out_specs=pl.BlockSpec((2, _BS, _D), lambda l, idx_ref, drop_ref: (l, 0, 0)),
    )
    return pl.pallas_call(
        _body,
        grid_spec=grid_spec,
        out_shape=jax.ShapeDtypeStruct((_L, _BS, _D), jnp.float32),
        compiler_params=pltpu.CompilerParams(
            dimension_semantics=("arbitrary",),
        ),
    )(index, drop_rand, x_t)


@jax.jit
def kernel(inputs, index, drop_rand):
    x_t = jnp.transpose(inputs, (1, 0, 2))
    out_t = _transposed_call(index, drop_rand, x_t)
    return jnp.transpose(out_t, (1, 0, 2))
